# Initial kernel scaffold; baseline (speedup 1.0000x reference)
import jax
import jax.numpy as jnp
from jax.experimental import pallas as pl


def kernel(x, pos_table):
    raise NotImplementedError("write your pallas kernel here")


# speedup vs baseline: 3.1745x; 3.1745x over previous
import jax
import jax.numpy as jnp
from jax.experimental import pallas as pl

_BLK_ROWS = 1024


def _copy_body(table_ref, out_ref):
    out_ref[...] = table_ref[...]


def kernel(x, pos_table):
    seqlen = x.shape[-1]
    embed = pos_table.shape[1]
    nblk = pl.cdiv(seqlen, _BLK_ROWS)
    return pl.pallas_call(
        _copy_body,
        grid=(nblk,),
        in_specs=[pl.BlockSpec((_BLK_ROWS, embed), lambda i: (i, 0))],
        out_specs=pl.BlockSpec((_BLK_ROWS, embed), lambda i: (i, 0)),
        out_shape=jax.ShapeDtypeStruct((seqlen, embed), pos_table.dtype),
    )(pos_table)


# speedup vs baseline: 3.3761x; 1.0635x over previous
import jax
import jax.numpy as jnp
from jax.experimental import pallas as pl
from jax.experimental.pallas import tpu as pltpu

_BLK_ROWS = 2048


def _copy_body(table_ref, out_ref):
    out_ref[...] = table_ref[...]


def kernel(x, pos_table):
    seqlen = x.shape[-1]
    embed = pos_table.shape[1]
    nblk = pl.cdiv(seqlen, _BLK_ROWS)
    return pl.pallas_call(
        _copy_body,
        grid=(nblk,),
        in_specs=[pl.BlockSpec((_BLK_ROWS, embed), lambda i: (i, 0))],
        out_specs=pl.BlockSpec((_BLK_ROWS, embed), lambda i: (i, 0)),
        out_shape=jax.ShapeDtypeStruct((seqlen, embed), pos_table.dtype),
        compiler_params=pltpu.CompilerParams(
            dimension_semantics=("parallel",),
        ),
    )(pos_table)
